# baseline (speedup 1.0000x reference)
import functools

import jax
import jax.numpy as jnp
from jax import lax
from jax.experimental import pallas as pl
from jax.experimental.pallas import tpu as pltpu

_TILE = 1024
_LANES = 128


def _row_runs(tile, src_rows, offset):
    idx = [min((1000 * r) // 1024 + offset, src_rows - 1) for r in range(tile)]
    runs = []
    start = 0
    for k in range(1, tile):
        if idx[k] != idx[k - 1] + 1:
            runs.append((idx[start], idx[start] + (k - start)))
            start = k
    runs.append((idx[start], idx[start] + (tile - start)))
    return runs


def _gather_rows(m, runs):
    return jnp.concatenate([m[s:e] for s, e in runs], axis=0)


def _fused_body(runs_a, runs_b, state_ref, action_ref, values_ref, out_ref):
    m = state_ref[...].astype(jnp.bfloat16)
    a_rows = _gather_rows(m, runs_a)
    b_rows = _gather_rows(m, runs_b)
    skew = 24
    r_iota = lax.broadcasted_iota(jnp.int32, (_TILE, 1024), 0)
    lane = lax.broadcasted_iota(jnp.int32, (_TILE, 1024), 1)
    phi = (-skew * r_iota) % 1024
    t = jnp.where(lane >= phi, a_rows, b_rows)
    for _ in range(skew // 8):
        t = pltpu.roll(t, 0, 1, stride=8, stride_axis=0)
    v = values_ref[...].astype(jnp.bfloat16)
    q = jnp.dot(t, v, preferred_element_type=jnp.float32)
    red = (action_ref[...] * q).reshape(_TILE // _LANES, _LANES, -1)
    out_ref[...] = jnp.sum(red, axis=2)


def kernel(state, action, values):
    batch, state_size = state.shape
    action_size = action.shape[1]
    flat_rows = batch * state_size // 1024
    block_rows = _TILE * state_size // 1024
    mview = state.reshape(flat_rows, 1024)
    values_pad = jnp.pad(values, ((0, 1024 - state_size), (0, 0)))
    sub = _TILE // _LANES
    grid = (batch // _TILE,)

    runs_a = _row_runs(_TILE, block_rows, 0)
    runs_b = _row_runs(_TILE, block_rows, 1)
    body = functools.partial(_fused_body, runs_a, runs_b)

    out = pl.pallas_call(
        body,
        grid=grid,
        in_specs=[
            pl.BlockSpec((block_rows, 1024), lambda i: (i, 0)),
            pl.BlockSpec((_TILE, action_size), lambda i: (i, 0)),
            pl.BlockSpec((1024, action_size), lambda i: (0, 0)),
        ],
        out_specs=pl.BlockSpec((sub, _LANES), lambda i: (i, 0)),
        out_shape=jax.ShapeDtypeStruct((batch // _LANES, _LANES), jnp.float32),
        compiler_params=pltpu.CompilerParams(
            dimension_semantics=("parallel",),
        ),
    )(mview, action, values_pad)
    return out.reshape(batch, 1)


# speedup vs baseline: 2.2164x; 2.2164x over previous
import jax
import jax.numpy as jnp
from jax.experimental import pallas as pl
from jax.experimental.pallas import tpu as pltpu

_TILE = 2048
_LANES = 128


def _fused_body(state_ref, action_ref, values_ref, out_ref):
    s = state_ref[...].astype(jnp.bfloat16)
    v = values_ref[...].astype(jnp.bfloat16)
    q = jnp.dot(s, v, preferred_element_type=jnp.float32)
    r = (action_ref[...] * q).reshape(_TILE // _LANES, _LANES, -1)
    out_ref[...] = jnp.sum(r, axis=2)


def kernel(state, action, values):
    batch, state_size = state.shape
    action_size = action.shape[1]
    sub = _TILE // _LANES
    grid = (batch // _TILE,)
    out = pl.pallas_call(
        _fused_body,
        grid=grid,
        in_specs=[
            pl.BlockSpec((_TILE, state_size), lambda i: (i, 0)),
            pl.BlockSpec((_TILE, action_size), lambda i: (i, 0)),
            pl.BlockSpec((state_size, action_size), lambda i: (0, 0)),
        ],
        out_specs=pl.BlockSpec((sub, _LANES), lambda i: (i, 0)),
        out_shape=jax.ShapeDtypeStruct((batch // _LANES, _LANES), jnp.float32),
        compiler_params=pltpu.CompilerParams(
            dimension_semantics=("parallel",),
        ),
    )(state, action, values)
    return out.reshape(batch, 1)
